# Initial kernel scaffold; baseline (speedup 1.0000x reference)
import jax
import jax.numpy as jnp
from jax.experimental import pallas as pl


def kernel(x, edge_index, y, gat0_Wl, gat0_Wr, gat0_att, gat0_b, gat1_Wl, gat1_Wr, gat1_att, gat1_b, gat2_Wl, gat2_Wr, gat2_att, gat2_b, tag0_W, tag0_b, tag1_W, tag1_b, tag2_W, tag2_b, lin_W, lin_b):
    raise NotImplementedError("write your pallas kernel here")


# speedup vs baseline: 1.0001x; 1.0001x over previous
import jax
import jax.numpy as jnp
from jax.experimental import pallas as pl


def _gatv2(x, src, dst, Wl, Wr, att, b):
    n = x.shape[0]
    loop = jnp.arange(n)
    s = jnp.concatenate([src, loop])
    d = jnp.concatenate([dst, loop])
    xl = x @ Wl.T
    xr = x @ Wr.T
    e = jax.nn.leaky_relu(xl[s] + xr[d], negative_slope=0.2)
    logit = e @ att
    m = jax.ops.segment_max(logit, d, num_segments=n)
    m = jnp.where(jnp.isfinite(m), m, 0.0)
    p = jnp.exp(logit - m[d])
    z = jax.ops.segment_sum(p, d, num_segments=n)
    alpha = p / (z[d] + 1e-16)
    out = jax.ops.segment_sum(xl[s] * alpha[:, None], d, num_segments=n)
    return out + b


def _tag(x, src, dst, W, b):
    n = x.shape[0]
    ones = jnp.ones(src.shape[0], dtype=x.dtype)
    deg = jax.ops.segment_sum(ones, dst, num_segments=n)
    dinv = jnp.where(deg > 0, jax.lax.rsqrt(jnp.maximum(deg, 1e-12)), 0.0)
    norm = dinv[src] * dinv[dst]
    out = x @ W[0].T
    h = x
    for k in range(1, W.shape[0]):
        h = jax.ops.segment_sum(h[src] * norm[:, None], dst, num_segments=n)
        out = out + h @ W[k].T
    return out + b


def kernel(x, edge_index, y, gat0_Wl, gat0_Wr, gat0_att, gat0_b, gat1_Wl, gat1_Wr, gat1_att, gat1_b, gat2_Wl, gat2_Wr, gat2_att, gat2_b, tag0_W, tag0_b, tag1_W, tag1_b, tag2_W, tag2_b, lin_W, lin_b):
    src, dst = edge_index[0], edge_index[1]
    gat = [(gat0_Wl, gat0_Wr, gat0_att, gat0_b), (gat1_Wl, gat1_Wr, gat1_att, gat1_b), (gat2_Wl, gat2_Wr, gat2_att, gat2_b)]
    tag = [(tag0_W, tag0_b), (tag1_W, tag1_b), (tag2_W, tag2_b)]
    x1 = x
    for i, (Wl, Wr, att, b) in enumerate(gat):
        x1 = _gatv2(x1, src, dst, Wl, Wr, att, b)
        if i < len(gat) - 1:
            x1 = jax.nn.relu(x1)
    x2 = x
    for i, (W, b) in enumerate(tag):
        x2 = _tag(x2, src, dst, W, b)
        if i < len(tag) - 1:
            x2 = jax.nn.relu(x2)
    xc = jnp.concatenate([x1, x2], axis=1)
    xo = jax.nn.relu(xc @ lin_W.T + lin_b)
    keep = jax.random.bernoulli(jax.random.key(1), 0.02, xo.shape)
    x_i = jnp.where(keep, xo / 0.02, 0.0)
    return jnp.where(y == 0.0, x_i, xo)


# speedup vs baseline: 5.0076x; 5.0072x over previous
import functools

import jax
import jax.numpy as jnp
from jax import lax
from jax.experimental import pallas as pl
from jax.experimental.pallas import tpu as pltpu
from jax.experimental.pallas import tpu_sc as plsc

_N = 10000
_NP = 10112
_E = 320000
_H = 128
_NC = 2
_NS = 16
_NW = _NC * _NS
_EPW = _E // _NW
_C = 80
_NCH = _EPW // _C
_NK = _NW * _NCH
_RPT = _NP // _NS


def _mesh():
    return plsc.VectorSubcoreMesh(core_axis_name="c", subcore_axis_name="s")


def _zero16():
    return jnp.zeros((16,), jnp.float32)


_GDN = lax.GatherDimensionNumbers(
    offset_dims=(), collapsed_slice_dims=(0,), start_index_map=(0,))


def _allsum16(v, i16):
    for k in (1, 2, 4, 8):
        perm = (i16 ^ k)[:, None]
        v = v + lax.gather(v, perm, _GDN, (1,),
                           mode=lax.GatherScatterMode.PROMISE_IN_BOUNDS,
                           unique_indices=True, indices_are_sorted=False)
    return v


def _zero_rows(ref, rows, width):
    def body(i, carry):
        for q in range(width // 16):
            ref[i, pl.ds(q * 16, 16)] = _zero16()
        return carry
    lax.fori_loop(0, rows, body, 0)


def _clear_shared(zsrc, sh, sid):
    base = sid * _RPT
    for off, ln in ((0, 80), (80, 80), (160, 80), (240, 80), (320, 80),
                    (400, 80), (480, 80), (560, 72)):
        pltpu.sync_copy(zsrc.at[pl.ds(0, ln)], sh.at[pl.ds(base + off, ln)])


def _export_shared(sh, out, cid, sid):
    base = sid * _RPT
    pltpu.sync_copy(sh.at[pl.ds(base, _RPT)], out.at[cid, pl.ds(base, _RPT)])


@functools.partial(
    pl.kernel,
    out_type=jax.ShapeDtypeStruct((_NC, _NP, 16), jnp.float32),
    mesh=_mesh(),
    scratch_types=[
        pltpu.VMEM((1, _C), jnp.int32),
        pltpu.VMEM((_C, 16), jnp.float32),
        pltpu.VMEM_SHARED((_NP, 16), jnp.float32),
    ],
)
def _sc_deg(dst3, out, dstr, onesb, deg_sh):
    cid = lax.axis_index("c")
    sid = lax.axis_index("s")
    w = cid * _NS + sid
    i16 = lax.iota(jnp.int32, 16)
    _zero_rows(onesb, _C, 16)
    _clear_shared(onesb, deg_sh, sid)
    one_row = jnp.where(i16 == 0, 1.0, 0.0)

    def ones_body(i, carry):
        onesb[i, pl.ds(0, 16)] = one_row
        return carry
    lax.fori_loop(0, _C, ones_body, 0)
    plsc.subcore_barrier()

    def chunk(j, carry):
        pltpu.sync_copy(dst3.at[w * _NCH + j], dstr)
        pltpu.sync_copy(onesb, deg_sh.at[dstr.at[0]], add=True)
        return carry
    lax.fori_loop(0, _NCH, chunk, 0)
    plsc.subcore_barrier()
    _export_shared(deg_sh, out, cid, sid)


@functools.partial(
    pl.kernel,
    out_type=jax.ShapeDtypeStruct((_NK, 1, _C), jnp.float32),
    mesh=_mesh(),
    scratch_types=[
        pltpu.VMEM((1, _C), jnp.int32),
        pltpu.VMEM((1, _C), jnp.int32),
        pltpu.VMEM((_C,), jnp.float32),
        pltpu.VMEM((_C,), jnp.float32),
        pltpu.VMEM((1, _C), jnp.float32),
        pltpu.SemaphoreType.DMA,
    ],
)
def _sc_norm(dinv, src3, dst3, out, srcr, dstr, av, bv, navb, sem):
    cid = lax.axis_index("c")
    sid = lax.axis_index("s")
    w = cid * _NS + sid

    def chunk(j, carry):
        k = w * _NCH + j
        pltpu.sync_copy(src3.at[k], srcr)
        pltpu.sync_copy(dst3.at[k], dstr)
        pltpu.async_copy(dinv.at[srcr.at[0]], av, sem).wait()
        pltpu.async_copy(dinv.at[dstr.at[0]], bv, sem).wait()
        for g in range(_C // 16):
            s = pl.ds(g * 16, 16)
            navb[0, s] = av[s] * bv[s]
        pltpu.sync_copy(navb, out.at[k])
        return carry
    lax.fori_loop(0, _NCH, chunk, 0)


@functools.partial(
    pl.kernel,
    out_type=(
        jax.ShapeDtypeStruct((_NK, 1, _C), jnp.float32),
        jax.ShapeDtypeStruct((_NC, _NP, 16), jnp.float32),
    ),
    mesh=_mesh(),
    scratch_types=[
        pltpu.VMEM((1, _C), jnp.int32),
        pltpu.VMEM((1, _C), jnp.int32),
        pltpu.VMEM((_C, _H), jnp.float32),
        pltpu.VMEM((_C, _H), jnp.float32),
        pltpu.VMEM((_H,), jnp.float32),
        pltpu.VMEM((_C,), jnp.float32),
        pltpu.VMEM((1, _C), jnp.float32),
        pltpu.VMEM((_C, 16), jnp.float32),
        pltpu.VMEM_SHARED((_NP, 16), jnp.float32),
        pltpu.SemaphoreType.DMA,
    ],
)
def _sc_gat_logits(xl, xr, cvals, att, src3, dst3, p_out, z_out,
                   srcr, dstr, xlb, xrb, attv, cb, pout, pzb, z_sh, sem):
    cid = lax.axis_index("c")
    sid = lax.axis_index("s")
    w = cid * _NS + sid
    _zero_rows(pzb, _C, 16)
    _clear_shared(pzb, z_sh, sid)
    plsc.subcore_barrier()

    pltpu.sync_copy(att, attv)
    attq = [attv[pl.ds(q * 16, 16)] for q in range(_H // 16)]
    i16 = lax.iota(jnp.int32, 16)

    def chunk(j, carry):
        k = w * _NCH + j
        pltpu.sync_copy(src3.at[k], srcr)
        pltpu.sync_copy(dst3.at[k], dstr)
        pltpu.async_copy(xl.at[srcr.at[0]], xlb, sem).wait()
        pltpu.async_copy(xr.at[dstr.at[0]], xrb, sem).wait()
        pltpu.async_copy(cvals.at[dstr.at[0]], cb, sem).wait()

        def group(g, gcarry):
            rb = g * 16
            lvec = _zero16()
            for rr in range(16):
                r = rb + rr
                acc = _zero16()
                for q in range(_H // 16):
                    s = pl.ds(q * 16, 16)
                    sv = xlb[r, s] + xrb[r, s]
                    ev = jnp.maximum(sv, 0.2 * sv)
                    acc = acc + attq[q] * ev
                lvec = jnp.where(i16 == rr, _allsum16(acc, i16), lvec)
            pvec = jnp.exp(lvec - cb[pl.ds(rb, 16)])
            pout[0, pl.ds(rb, 16)] = pvec
            for rr in range(16):
                pzb[rb + rr, pl.ds(0, 16)] = jnp.where(i16 == 0, pvec[rr], 0.0)
            return gcarry
        lax.fori_loop(0, _C // 16, group, 0)

        pltpu.sync_copy(pout, p_out.at[k])
        pltpu.sync_copy(pzb, z_sh.at[dstr.at[0]], add=True)
        return carry
    lax.fori_loop(0, _NCH, chunk, 0)
    plsc.subcore_barrier()
    _export_shared(z_sh, z_out, cid, sid)


@functools.partial(
    pl.kernel,
    out_type=(
        jax.ShapeDtypeStruct((_NC, _NP, 16), jnp.float32),
        jax.ShapeDtypeStruct((_NC, _NP, 16), jnp.float32),
    ),
    mesh=_mesh(),
    scratch_types=[
        pltpu.VMEM((1, _C), jnp.int32),
        pltpu.VMEM((1, _C), jnp.int32),
        pltpu.VMEM((_C,), jnp.float32),
        pltpu.VMEM((_C,), jnp.float32),
        pltpu.VMEM((_C,), jnp.float32),
        pltpu.VMEM((_C, 16), jnp.float32),
        pltpu.VMEM((_C, 16), jnp.float32),
        pltpu.VMEM((16,), jnp.float32),
        pltpu.VMEM_SHARED((_NP, 16), jnp.float32),
        pltpu.VMEM_SHARED((_NP, 16), jnp.float32),
        pltpu.SemaphoreType.DMA,
    ],
)
def _sc_gat2_edge(xlv, xrv, cvals, att16, src3, dst3, a_out, z_out,
                  srcr, dstr, xb, rb2, cb, pzb, qzb, attv, a_sh, z_sh, sem):
    cid = lax.axis_index("c")
    sid = lax.axis_index("s")
    w = cid * _NS + sid
    _zero_rows(pzb, _C, 16)
    _clear_shared(pzb, a_sh, sid)
    _clear_shared(pzb, z_sh, sid)
    plsc.subcore_barrier()

    pltpu.sync_copy(att16, attv)
    a16 = attv[pl.ds(0, 16)]
    i16 = lax.iota(jnp.int32, 16)

    def chunk(j, carry):
        k = w * _NCH + j
        pltpu.sync_copy(src3.at[k], srcr)
        pltpu.sync_copy(dst3.at[k], dstr)
        pltpu.async_copy(xlv.at[srcr.at[0]], xb, sem).wait()
        pltpu.async_copy(xrv.at[dstr.at[0]], rb2, sem).wait()
        pltpu.async_copy(cvals.at[dstr.at[0]], cb, sem).wait()
        for g in range(_C // 16):
            rb = g * 16
            s = pl.ds(rb, 16)
            sv = xb[s] + rb2[s]
            ev = jnp.maximum(sv, 0.2 * sv)
            pv = jnp.exp(a16 * ev - cb[s])
            qv = pv * xb[s]
            for rr in range(16):
                pzb[rb + rr, pl.ds(0, 16)] = jnp.where(i16 == 0, pv[rr], 0.0)
                qzb[rb + rr, pl.ds(0, 16)] = jnp.where(i16 == 0, qv[rr], 0.0)
        pltpu.sync_copy(pzb, z_sh.at[dstr.at[0]], add=True)
        pltpu.sync_copy(qzb, a_sh.at[dstr.at[0]], add=True)
        return carry
    lax.fori_loop(0, _NCH, chunk, 0)
    plsc.subcore_barrier()
    _export_shared(a_sh, a_out, cid, sid)
    _export_shared(z_sh, z_out, cid, sid)


@functools.partial(
    pl.kernel,
    out_type=jax.ShapeDtypeStruct((_NC, _NP, _H), jnp.float32),
    mesh=_mesh(),
    scratch_types=[
        pltpu.VMEM((1, _C), jnp.int32),
        pltpu.VMEM((1, _C), jnp.int32),
        pltpu.VMEM((1, _C), jnp.float32),
        pltpu.VMEM((_C, _H), jnp.float32),
        pltpu.VMEM((_C, _H), jnp.float32),
        pltpu.VMEM_SHARED((_NP, _H), jnp.float32),
        pltpu.SemaphoreType.DMA,
    ],
)
def _sc_wagg(tab, w3, src3, dst3, acc_out,
             srcr, dstr, wr, hb, outb, acc_sh, sem):
    cid = lax.axis_index("c")
    sid = lax.axis_index("s")
    w = cid * _NS + sid

    _zero_rows(outb, _C, _H)
    _clear_shared(outb, acc_sh, sid)
    plsc.subcore_barrier()

    def chunk(j, carry):
        k = w * _NCH + j
        pltpu.sync_copy(src3.at[k], srcr)
        pltpu.sync_copy(dst3.at[k], dstr)
        pltpu.sync_copy(w3.at[k], wr)
        pltpu.async_copy(tab.at[srcr.at[0]], hb, sem).wait()

        def group(g, gcarry):
            rb = g * 16
            nvv = wr[0, pl.ds(rb, 16)]
            for rr in range(16):
                r = rb + rr
                nv = nvv[rr]
                for q in range(_H // 16):
                    s = pl.ds(q * 16, 16)
                    outb[r, s] = hb[r, s] * nv
            return gcarry
        lax.fori_loop(0, _C // 16, group, 0)

        pltpu.sync_copy(outb, acc_sh.at[dstr.at[0]], add=True)
        return carry
    lax.fori_loop(0, _NCH, chunk, 0)
    plsc.subcore_barrier()
    _export_shared(acc_sh, acc_out, cid, sid)


def _mmT(a, b):
    return lax.dot_general(a, b, (((1,), (1,)), ((), ())),
                           preferred_element_type=jnp.float32)


def _gat_prep_body(x_ref, wl_ref, wr_ref, att_ref, xl_ref, xr_ref, c_ref):
    x = x_ref[...]
    xlv = _mmT(x, wl_ref[...])
    xrv = _mmT(x, wr_ref[...])
    s = xlv + xrv
    e = jnp.maximum(s, 0.2 * s)
    c_ref[...] = jnp.sum(e * att_ref[...][None, :], axis=1)
    xl_ref[...] = xlv
    xr_ref[...] = xrv


_gat_prep = pl.pallas_call(
    _gat_prep_body,
    out_shape=(
        jax.ShapeDtypeStruct((_N, _H), jnp.float32),
        jax.ShapeDtypeStruct((_N, _H), jnp.float32),
        jax.ShapeDtypeStruct((_N,), jnp.float32),
    ),
)


def _gat_post_body(a_ref, z_ref, xl_ref, b_ref, o_ref):
    a = a_ref[...]
    z = z_ref[...]
    num = a[0, :_N, :] + a[1, :_N, :] + xl_ref[...]
    den = z[0, :_N, 0] + z[1, :_N, 0] + 1.0
    v = num / den[:, None] + b_ref[...][None, :]
    o_ref[...] = jnp.maximum(v, 0.0)


_gat_post = pl.pallas_call(
    _gat_post_body,
    out_shape=jax.ShapeDtypeStruct((_N, _H), jnp.float32),
)


def _gat2_prep_body(x_ref, wl_ref, wr_ref, att_ref, xl_ref, xr_ref, c_ref):
    x = x_ref[...]
    xlv = _mmT(x, wl_ref[...])
    xrv = _mmT(x, wr_ref[...])
    s = xlv + xrv
    e = jnp.maximum(s, 0.2 * s)
    c_ref[...] = jnp.sum(e * att_ref[...][None, :], axis=1)
    xl_ref[...] = xlv[:, 0]
    xr_ref[...] = xrv[:, 0]


_gat2_prep = pl.pallas_call(
    _gat2_prep_body,
    out_shape=(
        jax.ShapeDtypeStruct((_N,), jnp.float32),
        jax.ShapeDtypeStruct((_N,), jnp.float32),
        jax.ShapeDtypeStruct((_N,), jnp.float32),
    ),
)


def _gat2_post_body(a_ref, z_ref, xl_ref, b_ref, o_ref):
    a = a_ref[...]
    z = z_ref[...]
    num = a[0, :_N, 0] + a[1, :_N, 0] + xl_ref[...]
    den = z[0, :_N, 0] + z[1, :_N, 0] + 1.0
    o_ref[...] = num / den + b_ref[...]


_gat2_post = pl.pallas_call(
    _gat2_post_body,
    out_shape=jax.ShapeDtypeStruct((_N,), jnp.float32),
)


def _dinv_body(d_ref, o_ref):
    d = d_ref[...]
    dv = d[0, :_N, 0] + d[1, :_N, 0]
    o_ref[...] = jnp.where(dv > 0.0, lax.rsqrt(jnp.maximum(dv, 1e-12)), 0.0)


_dinv = pl.pallas_call(
    _dinv_body,
    out_shape=jax.ShapeDtypeStruct((_N,), jnp.float32),
)


def _tag_init_body(x_ref, w_ref, o_ref):
    o_ref[...] = _mmT(x_ref[...], w_ref[...])


def _make_tag_init(co):
    return pl.pallas_call(
        _tag_init_body,
        out_shape=jax.ShapeDtypeStruct((_N, co), jnp.float32),
    )


def _tag_mid_body(a_ref, oa_ref, w_ref, h_ref, o_ref):
    a = a_ref[...]
    hv = a[0, :_N, :] + a[1, :_N, :]
    h_ref[...] = hv
    o_ref[...] = oa_ref[...] + _mmT(hv, w_ref[...])


def _make_tag_mid(co):
    return pl.pallas_call(
        _tag_mid_body,
        out_shape=(
            jax.ShapeDtypeStruct((_N, _H), jnp.float32),
            jax.ShapeDtypeStruct((_N, co), jnp.float32),
        ),
    )


def _tag_last_body_relu(a_ref, oa_ref, w_ref, b_ref, o_ref):
    a = a_ref[...]
    hv = a[0, :_N, :] + a[1, :_N, :]
    v = oa_ref[...] + _mmT(hv, w_ref[...]) + b_ref[...][None, :]
    o_ref[...] = jnp.maximum(v, 0.0)


def _tag_last_body(a_ref, oa_ref, w_ref, b_ref, o_ref):
    a = a_ref[...]
    hv = a[0, :_N, :] + a[1, :_N, :]
    o_ref[...] = oa_ref[...] + _mmT(hv, w_ref[...]) + b_ref[...][None, :]


def _make_tag_last(co, relu):
    return pl.pallas_call(
        _tag_last_body_relu if relu else _tag_last_body,
        out_shape=jax.ShapeDtypeStruct((_N, co), jnp.float32),
    )


def _final_body(x1_ref, x2_ref, y_ref, keep_ref, w_ref, b_ref, o_ref):
    wv = w_ref[...]
    xo = x1_ref[...] * wv[0:1, 0:1] + x2_ref[...] * wv[0:1, 1:2] + b_ref[...]
    xo = jnp.maximum(xo, 0.0)
    x_i = keep_ref[...] * (xo / 0.02)
    o_ref[...] = jnp.where(y_ref[...] == 0.0, x_i, xo)


_final = pl.pallas_call(
    _final_body,
    out_shape=jax.ShapeDtypeStruct((_N, 1), jnp.float32),
)


def kernel(x, edge_index, y, gat0_Wl, gat0_Wr, gat0_att, gat0_b, gat1_Wl,
           gat1_Wr, gat1_att, gat1_b, gat2_Wl, gat2_Wr, gat2_att, gat2_b,
           tag0_W, tag0_b, tag1_W, tag1_b, tag2_W, tag2_b, lin_W, lin_b):
    src3 = edge_index[0].reshape(_NK, 1, _C)
    dst3 = edge_index[1].reshape(_NK, 1, _C)

    deg = _sc_deg(dst3)
    dinv = _dinv(deg)
    norm3 = _sc_norm(dinv, src3, dst3)

    x1 = x
    for Wl, Wr, att, b in ((gat0_Wl, gat0_Wr, gat0_att, gat0_b),
                           (gat1_Wl, gat1_Wr, gat1_att, gat1_b)):
        xlv, xrv, cv = _gat_prep(x1, Wl, Wr, att)
        p3, z = _sc_gat_logits(xlv, xrv, cv, att, src3, dst3)
        acc = _sc_wagg(xlv, p3, src3, dst3)
        x1 = _gat_post(acc, z, xlv, b)

    xl1, xr1, c1 = _gat2_prep(x1, gat2_Wl, gat2_Wr, gat2_att)
    att16 = jnp.broadcast_to(gat2_att, (16,))
    a2, z2 = _sc_gat2_edge(xl1, xr1, c1, att16, src3, dst3)
    x1f = _gat2_post(a2, z2, xl1, gat2_b)

    x2 = x
    for li, (W, b) in enumerate(((tag0_W, tag0_b), (tag1_W, tag1_b),
                                 (tag2_W, tag2_b))):
        co = W.shape[1]
        out_acc = _make_tag_init(co)(x2, W[0])
        h = x2
        for k in range(1, 4):
            accp = _sc_wagg(h, norm3, src3, dst3)
            if k < 3:
                h, out_acc = _make_tag_mid(co)(accp, out_acc, W[k])
            else:
                x2 = _make_tag_last(co, relu=li < 2)(accp, out_acc, W[k], b)

    keepf = jax.random.bernoulli(
        jax.random.key(1), 0.02, (_N, 1)).astype(jnp.float32)
    return _final(x1f.reshape(_N, 1), x2, y, keepf, lin_W, lin_b)


# speedup vs baseline: 7.1156x; 1.4210x over previous
import functools

import jax
import jax.numpy as jnp
from jax import lax
from jax.experimental import pallas as pl
from jax.experimental.pallas import tpu as pltpu
from jax.experimental.pallas import tpu_sc as plsc

_N = 10000
_NP = 10112
_E = 320000
_H = 128
_NC = 2
_NS = 16
_NW = _NC * _NS
_EPW = _E // _NW
_C = 80
_NCH = _EPW // _C
_NK = _NW * _NCH
_RPT = _NP // _NS
_NB = 5
_NBLK = _NCH // _NB


def _mesh():
    return plsc.VectorSubcoreMesh(core_axis_name="c", subcore_axis_name="s")


def _zero16():
    return jnp.zeros((16,), jnp.float32)


_GDN = lax.GatherDimensionNumbers(
    offset_dims=(), collapsed_slice_dims=(0,), start_index_map=(0,))


def _allsum16(v, i16):
    for k in (1, 2, 4, 8):
        perm = (i16 ^ k)[:, None]
        v = v + lax.gather(v, perm, _GDN, (1,),
                           mode=lax.GatherScatterMode.PROMISE_IN_BOUNDS,
                           unique_indices=True, indices_are_sorted=False)
    return v


def _zero_rows(ref, rows, width):
    def body(i, carry):
        for q in range(width // 16):
            ref[i, pl.ds(q * 16, 16)] = _zero16()
        return carry
    lax.fori_loop(0, rows, body, 0)


def _clear_shared(zsrc, sh, sid):
    base = sid * _RPT
    for off, ln in ((0, 80), (80, 80), (160, 80), (240, 80), (320, 80),
                    (400, 80), (480, 80), (560, 72)):
        pltpu.sync_copy(zsrc.at[pl.ds(0, ln)], sh.at[pl.ds(base + off, ln)])


def _export_shared(sh, out, cid, sid):
    base = sid * _RPT
    pltpu.sync_copy(sh.at[pl.ds(base, _RPT)], out.at[cid, pl.ds(base, _RPT)])


@functools.partial(
    pl.kernel,
    out_type=jax.ShapeDtypeStruct((_NC, _NP, 16), jnp.float32),
    mesh=_mesh(),
    scratch_types=[
        pltpu.VMEM((1, _C), jnp.int32),
        pltpu.VMEM((_C, 16), jnp.float32),
        pltpu.VMEM_SHARED((_NP, 16), jnp.float32),
    ],
)
def _sc_deg(dst3, out, dstr, onesb, deg_sh):
    cid = lax.axis_index("c")
    sid = lax.axis_index("s")
    w = cid * _NS + sid
    i16 = lax.iota(jnp.int32, 16)
    _zero_rows(onesb, _C, 16)
    _clear_shared(onesb, deg_sh, sid)
    one_row = jnp.where(i16 == 0, 1.0, 0.0)

    def ones_body(i, carry):
        onesb[i, pl.ds(0, 16)] = one_row
        return carry
    lax.fori_loop(0, _C, ones_body, 0)
    plsc.subcore_barrier()

    def chunk(j, carry):
        pltpu.sync_copy(dst3.at[w * _NCH + j], dstr)
        pltpu.sync_copy(onesb, deg_sh.at[dstr.at[0]], add=True)
        return carry
    lax.fori_loop(0, _NCH, chunk, 0)
    plsc.subcore_barrier()
    _export_shared(deg_sh, out, cid, sid)


@functools.partial(
    pl.kernel,
    out_type=jax.ShapeDtypeStruct((_NK, 1, _C), jnp.float32),
    mesh=_mesh(),
    scratch_types=[
        pltpu.VMEM((1, _C), jnp.int32),
        pltpu.VMEM((1, _C), jnp.int32),
        pltpu.VMEM((_C,), jnp.float32),
        pltpu.VMEM((_C,), jnp.float32),
        pltpu.VMEM((1, _C), jnp.float32),
        pltpu.SemaphoreType.DMA,
    ],
)
def _sc_norm(dinv, src3, dst3, out, srcr, dstr, av, bv, navb, sem):
    cid = lax.axis_index("c")
    sid = lax.axis_index("s")
    w = cid * _NS + sid

    def chunk(j, carry):
        k = w * _NCH + j
        pltpu.sync_copy(src3.at[k], srcr)
        pltpu.sync_copy(dst3.at[k], dstr)
        pltpu.async_copy(dinv.at[srcr.at[0]], av, sem).wait()
        pltpu.async_copy(dinv.at[dstr.at[0]], bv, sem).wait()
        for g in range(_C // 16):
            s = pl.ds(g * 16, 16)
            navb[0, s] = av[s] * bv[s]
        pltpu.sync_copy(navb, out.at[k])
        return carry
    lax.fori_loop(0, _NCH, chunk, 0)


@functools.partial(
    pl.kernel,
    out_type=(
        jax.ShapeDtypeStruct((_NK, 1, _C), jnp.float32),
        jax.ShapeDtypeStruct((_NC, _NP, 16), jnp.float32),
    ),
    mesh=_mesh(),
    scratch_types=[
        pltpu.VMEM((1, _C), jnp.int32),
        pltpu.VMEM((1, _C), jnp.int32),
        pltpu.VMEM((_C, _H), jnp.float32),
        pltpu.VMEM((_C, _H), jnp.float32),
        pltpu.VMEM((_C,), jnp.float32),
        pltpu.VMEM((_H,), jnp.float32),
        pltpu.VMEM((1, _C), jnp.float32),
        pltpu.VMEM((_C, 16), jnp.float32),
        pltpu.VMEM_SHARED((_NP, 16), jnp.float32),
        pltpu.SemaphoreType.DMA,
        pltpu.SemaphoreType.DMA,
    ],
)
def _sc_gat_logits(xl, xr, cvals, att, src3, dst3, p_out, z_out,
                   srcr, dstr, xlb, xrb, cb,
                   attv, pout, pzb, z_sh, isem, gsem):
    cid = lax.axis_index("c")
    sid = lax.axis_index("s")
    w = cid * _NS + sid
    _zero_rows(pzb, _C, 16)
    _clear_shared(pzb, z_sh, sid)
    plsc.subcore_barrier()

    pltpu.sync_copy(att, attv)
    attq = [attv[pl.ds(q * 16, 16)] for q in range(_H // 16)]
    i16 = lax.iota(jnp.int32, 16)

    def chunk(j, carry):
        k = w * _NCH + j
        d0a = pltpu.async_copy(src3.at[k], srcr, isem)
        d0b = pltpu.async_copy(dst3.at[k], dstr, isem)
        d0a.wait()
        d1 = pltpu.async_copy(xl.at[srcr.at[0]], xlb, gsem)
        d0b.wait()
        d2 = pltpu.async_copy(xr.at[dstr.at[0]], xrb, gsem)
        d3 = pltpu.async_copy(cvals.at[dstr.at[0]], cb, gsem)
        d1.wait()
        d2.wait()
        d3.wait()

        def group(g, gcarry):
            rb = g * 16
            lvec = _zero16()
            for rr in range(16):
                r = rb + rr
                acc = _zero16()
                for q in range(_H // 16):
                    s = pl.ds(q * 16, 16)
                    sv = xlb[r, s] + xrb[r, s]
                    ev = jnp.maximum(sv, 0.2 * sv)
                    acc = acc + attq[q] * ev
                lvec = jnp.where(i16 == rr, _allsum16(acc, i16), lvec)
            pvec = jnp.exp(lvec - cb[pl.ds(rb, 16)])
            pout[0, pl.ds(rb, 16)] = pvec
            for rr in range(16):
                pzb[rb + rr, pl.ds(0, 16)] = jnp.where(
                    i16 == 0, pvec[rr], 0.0)
            return gcarry
        lax.fori_loop(0, _C // 16, group, 0)

        pltpu.sync_copy(pout, p_out.at[k])
        pltpu.sync_copy(pzb, z_sh.at[dstr.at[0]], add=True)
        return carry
    lax.fori_loop(0, _NCH, chunk, 0)
    plsc.subcore_barrier()
    _export_shared(z_sh, z_out, cid, sid)


@functools.partial(
    pl.kernel,
    out_type=(
        jax.ShapeDtypeStruct((_NC, _NP, 16), jnp.float32),
        jax.ShapeDtypeStruct((_NC, _NP, 16), jnp.float32),
    ),
    mesh=_mesh(),
    scratch_types=[
        pltpu.VMEM((1, _C), jnp.int32),
        pltpu.VMEM((1, _C), jnp.int32),
        pltpu.VMEM((_C,), jnp.float32),
        pltpu.VMEM((_C,), jnp.float32),
        pltpu.VMEM((_C,), jnp.float32),
        pltpu.VMEM((_C, 16), jnp.float32),
        pltpu.VMEM((_C, 16), jnp.float32),
        pltpu.VMEM((16,), jnp.float32),
        pltpu.VMEM_SHARED((_NP, 16), jnp.float32),
        pltpu.VMEM_SHARED((_NP, 16), jnp.float32),
        pltpu.SemaphoreType.DMA,
    ],
)
def _sc_gat2_edge(xlv, xrv, cvals, att16, src3, dst3, a_out, z_out,
                  srcr, dstr, xb, rb2, cb, pzb, qzb, attv, a_sh, z_sh, sem):
    cid = lax.axis_index("c")
    sid = lax.axis_index("s")
    w = cid * _NS + sid
    _zero_rows(pzb, _C, 16)
    _clear_shared(pzb, a_sh, sid)
    _clear_shared(pzb, z_sh, sid)
    plsc.subcore_barrier()

    pltpu.sync_copy(att16, attv)
    a16 = attv[pl.ds(0, 16)]
    i16 = lax.iota(jnp.int32, 16)

    def chunk(j, carry):
        k = w * _NCH + j
        pltpu.sync_copy(src3.at[k], srcr)
        pltpu.sync_copy(dst3.at[k], dstr)
        pltpu.async_copy(xlv.at[srcr.at[0]], xb, sem).wait()
        pltpu.async_copy(xrv.at[dstr.at[0]], rb2, sem).wait()
        pltpu.async_copy(cvals.at[dstr.at[0]], cb, sem).wait()
        for g in range(_C // 16):
            rb = g * 16
            s = pl.ds(rb, 16)
            sv = xb[s] + rb2[s]
            ev = jnp.maximum(sv, 0.2 * sv)
            pv = jnp.exp(a16 * ev - cb[s])
            qv = pv * xb[s]
            for rr in range(16):
                pzb[rb + rr, pl.ds(0, 16)] = jnp.where(i16 == 0, pv[rr], 0.0)
                qzb[rb + rr, pl.ds(0, 16)] = jnp.where(i16 == 0, qv[rr], 0.0)
        pltpu.sync_copy(pzb, z_sh.at[dstr.at[0]], add=True)
        pltpu.sync_copy(qzb, a_sh.at[dstr.at[0]], add=True)
        return carry
    lax.fori_loop(0, _NCH, chunk, 0)
    plsc.subcore_barrier()
    _export_shared(a_sh, a_out, cid, sid)
    _export_shared(z_sh, z_out, cid, sid)


@functools.partial(
    pl.kernel,
    out_type=jax.ShapeDtypeStruct((_NC, _NP, _H), jnp.float32),
    mesh=_mesh(),
    scratch_types=[
        pltpu.VMEM((1, _C), jnp.int32),
        pltpu.VMEM((1, _C), jnp.int32),
        pltpu.VMEM((1, _C), jnp.float32),
        pltpu.VMEM((_C, _H), jnp.float32),
        pltpu.VMEM_SHARED((_NP, _H), jnp.float32),
        pltpu.SemaphoreType.DMA,
        pltpu.SemaphoreType.DMA,
    ],
)
def _sc_wagg(tab, w3, src3, dst3, acc_out,
             srcr, dstr, wr, hb, acc_sh, isem, gsem):
    cid = lax.axis_index("c")
    sid = lax.axis_index("s")
    w = cid * _NS + sid

    _zero_rows(hb, _C, _H)
    _clear_shared(hb, acc_sh, sid)
    plsc.subcore_barrier()

    def chunk(j, carry):
        k = w * _NCH + j
        d0a = pltpu.async_copy(src3.at[k], srcr, isem)
        d0b = pltpu.async_copy(dst3.at[k], dstr, isem)
        d0c = pltpu.async_copy(w3.at[k], wr, isem)
        d0a.wait()
        dg = pltpu.async_copy(tab.at[srcr.at[0]], hb, gsem)
        d0b.wait()
        d0c.wait()
        dg.wait()

        def group2(g, gcarry):
            rb = g * 16
            nvv = wr[0, pl.ds(rb, 16)]
            for rr in range(16):
                r = rb + rr
                nv = nvv[rr]
                for q in range(_H // 16):
                    s = pl.ds(q * 16, 16)
                    hb[r, s] = hb[r, s] * nv
            return gcarry
        lax.fori_loop(0, _C // 16, group2, 0)

        pltpu.sync_copy(hb, acc_sh.at[dstr.at[0]], add=True)
        return carry
    lax.fori_loop(0, _NCH, chunk, 0)
    plsc.subcore_barrier()
    _export_shared(acc_sh, acc_out, cid, sid)


def _mmT(a, b):
    return lax.dot_general(a, b, (((1,), (1,)), ((), ())),
                           preferred_element_type=jnp.float32)


def _gat_prep_body(x_ref, wl_ref, wr_ref, att_ref, xl_ref, xr_ref, c_ref):
    x = x_ref[...]
    xlv = _mmT(x, wl_ref[...])
    xrv = _mmT(x, wr_ref[...])
    s = xlv + xrv
    e = jnp.maximum(s, 0.2 * s)
    c_ref[...] = jnp.sum(e * att_ref[...][None, :], axis=1)
    xl_ref[...] = xlv
    xr_ref[...] = xrv


_gat_prep = pl.pallas_call(
    _gat_prep_body,
    out_shape=(
        jax.ShapeDtypeStruct((_N, _H), jnp.float32),
        jax.ShapeDtypeStruct((_N, _H), jnp.float32),
        jax.ShapeDtypeStruct((_N,), jnp.float32),
    ),
)


def _gat_post_body(a_ref, z_ref, xl_ref, b_ref, o_ref):
    a = a_ref[...]
    z = z_ref[...]
    num = a[0, :_N, :] + a[1, :_N, :] + xl_ref[...]
    den = z[0, :_N, 0] + z[1, :_N, 0] + 1.0
    v = num / den[:, None] + b_ref[...][None, :]
    o_ref[...] = jnp.maximum(v, 0.0)


_gat_post = pl.pallas_call(
    _gat_post_body,
    out_shape=jax.ShapeDtypeStruct((_N, _H), jnp.float32),
)


def _gat2_prep_body(x_ref, wl_ref, wr_ref, att_ref, xl_ref, xr_ref, c_ref):
    x = x_ref[...]
    xlv = _mmT(x, wl_ref[...])
    xrv = _mmT(x, wr_ref[...])
    s = xlv + xrv
    e = jnp.maximum(s, 0.2 * s)
    c_ref[...] = jnp.sum(e * att_ref[...][None, :], axis=1)
    xl_ref[...] = xlv[:, 0]
    xr_ref[...] = xrv[:, 0]


_gat2_prep = pl.pallas_call(
    _gat2_prep_body,
    out_shape=(
        jax.ShapeDtypeStruct((_N,), jnp.float32),
        jax.ShapeDtypeStruct((_N,), jnp.float32),
        jax.ShapeDtypeStruct((_N,), jnp.float32),
    ),
)


def _gat2_post_body(a_ref, z_ref, xl_ref, b_ref, o_ref):
    a = a_ref[...]
    z = z_ref[...]
    num = a[0, :_N, 0] + a[1, :_N, 0] + xl_ref[...]
    den = z[0, :_N, 0] + z[1, :_N, 0] + 1.0
    o_ref[...] = num / den + b_ref[...]


_gat2_post = pl.pallas_call(
    _gat2_post_body,
    out_shape=jax.ShapeDtypeStruct((_N,), jnp.float32),
)


def _dinv_body(d_ref, o_ref):
    d = d_ref[...]
    dv = d[0, :_N, 0] + d[1, :_N, 0]
    o_ref[...] = jnp.where(dv > 0.0, lax.rsqrt(jnp.maximum(dv, 1e-12)), 0.0)


_dinv = pl.pallas_call(
    _dinv_body,
    out_shape=jax.ShapeDtypeStruct((_N,), jnp.float32),
)


def _tag_init_body(x_ref, w_ref, o_ref):
    o_ref[...] = _mmT(x_ref[...], w_ref[...])


def _make_tag_init(co):
    return pl.pallas_call(
        _tag_init_body,
        out_shape=jax.ShapeDtypeStruct((_N, co), jnp.float32),
    )


def _tag_mid_body(a_ref, oa_ref, w_ref, h_ref, o_ref):
    a = a_ref[...]
    hv = a[0, :_N, :] + a[1, :_N, :]
    h_ref[...] = hv
    o_ref[...] = oa_ref[...] + _mmT(hv, w_ref[...])


def _make_tag_mid(co):
    return pl.pallas_call(
        _tag_mid_body,
        out_shape=(
            jax.ShapeDtypeStruct((_N, _H), jnp.float32),
            jax.ShapeDtypeStruct((_N, co), jnp.float32),
        ),
    )


def _tag_last_body_relu(a_ref, oa_ref, w_ref, b_ref, o_ref):
    a = a_ref[...]
    hv = a[0, :_N, :] + a[1, :_N, :]
    v = oa_ref[...] + _mmT(hv, w_ref[...]) + b_ref[...][None, :]
    o_ref[...] = jnp.maximum(v, 0.0)


def _tag_last_body(a_ref, oa_ref, w_ref, b_ref, o_ref):
    a = a_ref[...]
    hv = a[0, :_N, :] + a[1, :_N, :]
    o_ref[...] = oa_ref[...] + _mmT(hv, w_ref[...]) + b_ref[...][None, :]


def _make_tag_last(co, relu):
    return pl.pallas_call(
        _tag_last_body_relu if relu else _tag_last_body,
        out_shape=jax.ShapeDtypeStruct((_N, co), jnp.float32),
    )


def _final_body(x1_ref, x2_ref, y_ref, keep_ref, w_ref, b_ref, o_ref):
    wv = w_ref[...]
    xo = x1_ref[...] * wv[0:1, 0:1] + x2_ref[...] * wv[0:1, 1:2] + b_ref[...]
    xo = jnp.maximum(xo, 0.0)
    x_i = keep_ref[...] * (xo / 0.02)
    o_ref[...] = jnp.where(y_ref[...] == 0.0, x_i, xo)


_final = pl.pallas_call(
    _final_body,
    out_shape=jax.ShapeDtypeStruct((_N, 1), jnp.float32),
)


def kernel(x, edge_index, y, gat0_Wl, gat0_Wr, gat0_att, gat0_b, gat1_Wl,
           gat1_Wr, gat1_att, gat1_b, gat2_Wl, gat2_Wr, gat2_att, gat2_b,
           tag0_W, tag0_b, tag1_W, tag1_b, tag2_W, tag2_b, lin_W, lin_b):
    src3 = edge_index[0].reshape(_NK, 1, _C)
    dst3 = edge_index[1].reshape(_NK, 1, _C)

    deg = _sc_deg(dst3)
    dinv = _dinv(deg)
    norm3 = _sc_norm(dinv, src3, dst3)

    x1 = x
    for Wl, Wr, att, b in ((gat0_Wl, gat0_Wr, gat0_att, gat0_b),
                           (gat1_Wl, gat1_Wr, gat1_att, gat1_b)):
        xlv, xrv, cv = _gat_prep(x1, Wl, Wr, att)
        p3, z = _sc_gat_logits(xlv, xrv, cv, att, src3, dst3)
        acc = _sc_wagg(xlv, p3, src3, dst3)
        x1 = _gat_post(acc, z, xlv, b)

    xl1, xr1, c1 = _gat2_prep(x1, gat2_Wl, gat2_Wr, gat2_att)
    att16 = jnp.broadcast_to(gat2_att, (16,))
    a2, z2 = _sc_gat2_edge(xl1, xr1, c1, att16, src3, dst3)
    x1f = _gat2_post(a2, z2, xl1, gat2_b)

    x2 = x
    for li, (W, b) in enumerate(((tag0_W, tag0_b), (tag1_W, tag1_b),
                                 (tag2_W, tag2_b))):
        co = W.shape[1]
        out_acc = _make_tag_init(co)(x2, W[0])
        h = x2
        for k in range(1, 4):
            accp = _sc_wagg(h, norm3, src3, dst3)
            if k < 3:
                h, out_acc = _make_tag_mid(co)(accp, out_acc, W[k])
            else:
                x2 = _make_tag_last(co, relu=li < 2)(accp, out_acc, W[k], b)

    keepf = jax.random.bernoulli(
        jax.random.key(1), 0.02, (_N, 1)).astype(jnp.float32)
    return _final(x1f.reshape(_N, 1), x2, y, keepf, lin_W, lin_b)
